# baseline (speedup 1.0000x reference)
import functools

import jax
import jax.numpy as jnp
from jax import lax
from jax.experimental import pallas as pl
from jax.experimental.pallas import tpu as pltpu
from jax.experimental.pallas import tpu_sc as plsc

B, S, H = 1, 2048, 768
I = 1536
E = 8
K = 2
T = B * S
P = K * T

BLK = 128
NB = P // BLK + E
P_MAX = NB * BLK

NW = 32
DCH = P // NW
GCH = T // NW

BT = 512
NT = T // BT


def _router_kernel(x_ref, wg_ref, pos_ref, w01_ref, bmap_ref, bact_ref,
                   aux_ref):
    x = x_ref[...]
    wg = wg_ref[...]
    logits = lax.dot_general(x, wg, (((1,), (1,)), ((), ())),
                             preferred_element_type=jnp.float32)
    m = jnp.max(logits, axis=1, keepdims=True)
    ex = jnp.exp(logits - m)
    denom = jnp.sum(ex, axis=1, keepdims=True)
    p = ex / denom

    i1 = jnp.argmax(p, axis=1)
    cols = lax.broadcasted_iota(jnp.int32, (T, E), 1)
    oh1 = (cols == i1[:, None]).astype(jnp.float32)
    v1 = jnp.sum(p * oh1, axis=1, keepdims=True)
    p2 = jnp.where(cols == i1[:, None], -jnp.inf, p)
    i2 = jnp.argmax(p2, axis=1)
    oh2 = (cols == i2[:, None]).astype(jnp.float32)
    v2 = jnp.sum(p * oh2, axis=1, keepdims=True)

    s = v1 + v2 + 1e-9
    w01_ref[...] = jnp.concatenate([v1 / s, v2 / s], axis=1)

    oh = jnp.concatenate([oh1, oh2], axis=0)
    c = oh
    sh = 1
    while sh < P:
        c = c + jnp.concatenate(
            [jnp.zeros((sh, E), jnp.float32), c[:P - sh]], axis=0)
        sh *= 2
    rank = jnp.sum((c - oh) * oh, axis=1)
    counts = c[P - 1:P, :]
    pc = jnp.ceil(counts / BLK) * BLK
    cp = pc
    sh = 1
    while sh < E:
        cp = cp + jnp.concatenate(
            [jnp.zeros((1, sh), jnp.float32), cp[:, :E - sh]], axis=1)
        sh *= 2
    excl = cp - pc
    off = jnp.sum(oh * excl, axis=1)
    pos_ref[...] = (off + rank).astype(jnp.int32)[None, :]

    total_padded = cp[:, E - 1:E]
    istart = lax.broadcasted_iota(
        jnp.int32, (NB, 1), 0).astype(jnp.float32) * BLK
    be_raw = jnp.sum((istart >= cp).astype(jnp.int32), axis=1)
    bmap_ref[...] = jnp.minimum(be_raw, E - 1)[None, :]
    bact_ref[...] = (istart[:, 0] < total_padded[0, 0]).astype(
        jnp.int32)[None, :]

    tpe = counts[0] / (T * K + 1e-9)
    avg_p = jnp.mean(p, axis=0)
    lb = E * jnp.sum(tpe * avg_p)
    lse = m[:, 0] + jnp.log(denom[:, 0])
    z_loss = jnp.mean(jnp.square(lse)) * 0.001
    entropy = jnp.mean(-jnp.sum(p * jnp.log(p + 1e-9), axis=1))
    ent_loss = (jnp.log(jnp.float32(E)) - entropy) * 0.01
    usage = jnp.mean((tpe > 0.01).astype(jnp.float32))
    util_loss = (1.0 - usage) * 0.1
    aux = lb + z_loss + ent_loss + util_loss
    aux_ref[...] = jnp.broadcast_to(aux[None, None], (1, 1))


def _router(x, w_gate):
    return pl.pallas_call(
        _router_kernel,
        out_shape=(
            jax.ShapeDtypeStruct((1, P), jnp.int32),
            jax.ShapeDtypeStruct((T, 2), jnp.float32),
            jax.ShapeDtypeStruct((1, NB), jnp.int32),
            jax.ShapeDtypeStruct((1, NB), jnp.int32),
            jax.ShapeDtypeStruct((1, 1), jnp.float32),
        ),
    )(x, w_gate)


@functools.lru_cache(maxsize=None)
def _sc_dispatch_call():
    mesh = plsc.VectorSubcoreMesh(core_axis_name="c", subcore_axis_name="s")

    @functools.partial(
        pl.kernel,
        out_type=jax.ShapeDtypeStruct((P_MAX, H), jnp.float32),
        mesh=mesh,
        scratch_types=[
            pltpu.VMEM((DCH,), jnp.int32),
            pltpu.VMEM((DCH, H), jnp.float32),
            pltpu.SemaphoreType.DMA,
        ],
    )
    def _sc_dispatch(x_hbm, pos_hbm, xs_hbm, pos_v, rows_v, sem):
        wid = lax.axis_index("s") * 2 + lax.axis_index("c")
        base = wid * DCH
        tok_base = lax.rem(base, T)
        pltpu.sync_copy(pos_hbm.at[pl.ds(base, DCH)], pos_v)
        pltpu.sync_copy(x_hbm.at[pl.ds(tok_base, DCH)], rows_v)
        pltpu.async_copy(rows_v, xs_hbm.at[pos_v], sem).wait()

    return _sc_dispatch


@functools.lru_cache(maxsize=None)
def _sc_gatherback_call():
    mesh = plsc.VectorSubcoreMesh(core_axis_name="c", subcore_axis_name="s")

    @functools.partial(
        pl.kernel,
        out_type=(
            jax.ShapeDtypeStruct((T, H), jnp.float32),
            jax.ShapeDtypeStruct((T, H), jnp.float32),
        ),
        mesh=mesh,
        scratch_types=[
            pltpu.VMEM((GCH,), jnp.int32),
            pltpu.VMEM((GCH,), jnp.int32),
            pltpu.VMEM((GCH, H), jnp.float32),
            pltpu.VMEM((GCH, H), jnp.float32),
            pltpu.SemaphoreType.DMA,
        ],
    )
    def _sc_gatherback(ys_hbm, pos_hbm, y0_hbm, y1_hbm, p0_v, p1_v, r0_v,
                       r1_v, sem):
        wid = lax.axis_index("s") * 2 + lax.axis_index("c")
        base = wid * GCH
        pltpu.sync_copy(pos_hbm.at[pl.ds(base, GCH)], p0_v)
        pltpu.sync_copy(pos_hbm.at[pl.ds(T + base, GCH)], p1_v)
        cp0 = pltpu.async_copy(ys_hbm.at[p0_v], r0_v, sem)
        cp1 = pltpu.async_copy(ys_hbm.at[p1_v], r1_v, sem)
        cp0.wait()
        cp1.wait()
        pltpu.sync_copy(r0_v, y0_hbm.at[pl.ds(base, GCH)])
        pltpu.sync_copy(r1_v, y1_hbm.at[pl.ds(base, GCH)])

    return _sc_gatherback


def _xs_idx(i, bm, ba):
    return (jnp.where(ba[i] == 1, i, 0), 0)


def _w_idx(i, bm, ba):
    return (bm[i], 0, 0)


def _out_idx(i, bm, ba):
    return (jnp.where(ba[i] == 1, i, NB - 1), 0)


def _gmm_kernel(bmap_ref, bact_ref, xs_ref, wg_ref, wu_ref, wd_ref, out_ref):
    i = pl.program_id(0)

    @pl.when(bact_ref[i] == 1)
    def _():
        x = xs_ref[...]
        g = lax.dot_general(x, wg_ref[0], (((1,), (1,)), ((), ())),
                            preferred_element_type=jnp.float32)
        u = lax.dot_general(x, wu_ref[0], (((1,), (1,)), ((), ())),
                            preferred_element_type=jnp.float32)
        a = g * jax.nn.sigmoid(g) * u
        out_ref[...] = lax.dot_general(a, wd_ref[0], (((1,), (1,)), ((), ())),
                                       preferred_element_type=jnp.float32)


def _gmm(xs, Wg, Wu, Wd, bmap, bact):
    return pl.pallas_call(
        _gmm_kernel,
        grid_spec=pltpu.PrefetchScalarGridSpec(
            num_scalar_prefetch=2,
            grid=(NB,),
            in_specs=[
                pl.BlockSpec((BLK, H), _xs_idx),
                pl.BlockSpec((1, I, H), _w_idx),
                pl.BlockSpec((1, I, H), _w_idx),
                pl.BlockSpec((1, H, I), _w_idx),
            ],
            out_specs=pl.BlockSpec((BLK, H), _out_idx),
        ),
        out_shape=jax.ShapeDtypeStruct((P_MAX, H), jnp.float32),
        compiler_params=pltpu.CompilerParams(
            dimension_semantics=("arbitrary",),
        ),
    )(bmap, bact, xs, Wg, Wu, Wd)


def _shared_kernel(x_ref, wsg_ref, wsu_ref, wsd_ref, sg_ref, out_ref):
    x = x_ref[...]
    g = lax.dot_general(x, wsg_ref[...], (((1,), (1,)), ((), ())),
                        preferred_element_type=jnp.float32)
    u = lax.dot_general(x, wsu_ref[...], (((1,), (1,)), ((), ())),
                        preferred_element_type=jnp.float32)
    a = g * jax.nn.sigmoid(g) * u
    y = lax.dot_general(a, wsd_ref[...], (((1,), (1,)), ((), ())),
                        preferred_element_type=jnp.float32)
    out_ref[...] = y * jax.nn.sigmoid(sg_ref[0, 0])


def _shared(x, Wsg, Wsu, Wsd, sg):
    return pl.pallas_call(
        _shared_kernel,
        grid=(NT,),
        in_specs=[
            pl.BlockSpec((BT, H), lambda t: (t, 0)),
            pl.BlockSpec((I, H), lambda t: (0, 0)),
            pl.BlockSpec((I, H), lambda t: (0, 0)),
            pl.BlockSpec((H, I), lambda t: (0, 0)),
            pl.BlockSpec((1, 1), lambda t: (0, 0)),
        ],
        out_specs=pl.BlockSpec((BT, H), lambda t: (t, 0)),
        out_shape=jax.ShapeDtypeStruct((T, H), jnp.float32),
        compiler_params=pltpu.CompilerParams(
            dimension_semantics=("parallel",),
        ),
    )(x, Wsg, Wsu, Wsd, sg)


def _combine_kernel(y0_ref, y1_ref, sh_ref, w_ref, out_ref):
    w = w_ref[...]
    out_ref[...] = (y0_ref[...] * w[:, 0:1] + y1_ref[...] * w[:, 1:2]
                    + sh_ref[...])


def _combine(y0, y1, shr, w01):
    return pl.pallas_call(
        _combine_kernel,
        grid=(NT,),
        in_specs=[
            pl.BlockSpec((BT, H), lambda t: (t, 0)),
            pl.BlockSpec((BT, H), lambda t: (t, 0)),
            pl.BlockSpec((BT, H), lambda t: (t, 0)),
            pl.BlockSpec((BT, 2), lambda t: (t, 0)),
        ],
        out_specs=pl.BlockSpec((BT, H), lambda t: (t, 0)),
        out_shape=jax.ShapeDtypeStruct((T, H), jnp.float32),
        compiler_params=pltpu.CompilerParams(
            dimension_semantics=("parallel",),
        ),
    )(y0, y1, shr, w01)


def kernel(hidden_states, W_gate, Wg, Wu, Wd, Wsg, Wsu, Wsd, shared_gate):
    x = hidden_states.reshape(T, H)
    pos, w01, bmap, bact, aux = _router(x, W_gate)
    pos1d = pos.reshape(P)
    xs = _sc_dispatch_call()(x, pos1d)
    ys = _gmm(xs, Wg, Wu, Wd, bmap.reshape(NB), bact.reshape(NB))
    shr = _shared(x, Wsg, Wsu, Wsd, shared_gate.reshape(1, 1))
    y0, y1 = _sc_gatherback_call()(ys, pos1d)
    out = _combine(y0, y1, shr, w01)
    return out.reshape(B, S, H), aux[0, 0]


# speedup vs baseline: 1.2924x; 1.2924x over previous
import functools

import jax
import jax.numpy as jnp
from jax import lax
from jax.experimental import pallas as pl
from jax.experimental.pallas import tpu as pltpu
from jax.experimental.pallas import tpu_sc as plsc

B, S, H = 1, 2048, 768
I = 1536
E = 8
K = 2
T = B * S
P = K * T

BLK = 256
NB = P // BLK + E
P_MAX = NB * BLK

NW = 32
DCH = P // NW
GCH = T // NW

BT = 512
NT = T // BT


def _router_kernel(x_ref, wg_ref, pos_ref, w01_ref, bmap_ref, bact_ref,
                   aux_ref):
    x = x_ref[...]
    wg = wg_ref[...]
    logits = lax.dot_general(x, wg, (((1,), (1,)), ((), ())),
                             preferred_element_type=jnp.float32)
    m = jnp.max(logits, axis=1, keepdims=True)
    ex = jnp.exp(logits - m)
    denom = jnp.sum(ex, axis=1, keepdims=True)
    p = ex / denom

    i1 = jnp.argmax(p, axis=1)
    cols = lax.broadcasted_iota(jnp.int32, (T, E), 1)
    oh1 = (cols == i1[:, None]).astype(jnp.float32)
    v1 = jnp.sum(p * oh1, axis=1, keepdims=True)
    p2 = jnp.where(cols == i1[:, None], -jnp.inf, p)
    i2 = jnp.argmax(p2, axis=1)
    oh2 = (cols == i2[:, None]).astype(jnp.float32)
    v2 = jnp.sum(p * oh2, axis=1, keepdims=True)

    s = v1 + v2 + 1e-9
    w01_ref[...] = jnp.concatenate([v1 / s, v2 / s], axis=1)

    rows = lax.broadcasted_iota(jnp.int32, (E, T), 0)
    oh1t = (rows == i1[None, :]).astype(jnp.float32)
    oh2t = (rows == i2[None, :]).astype(jnp.float32)

    def _lane_cumsum_excl(a):
        c = a
        sh = 1
        while sh < T:
            c = c + jnp.concatenate(
                [jnp.zeros((E, sh), jnp.float32), c[:, :T - sh]], axis=1)
            sh *= 2
        return c - a

    c1 = _lane_cumsum_excl(oh1t)
    c2 = _lane_cumsum_excl(oh2t)
    cnt1 = jnp.sum(oh1t, axis=1, keepdims=True)
    counts = cnt1 + jnp.sum(oh2t, axis=1, keepdims=True)
    pc = jnp.ceil(counts / BLK) * BLK
    cp = pc
    sh = 1
    while sh < E:
        cp = cp + jnp.concatenate(
            [jnp.zeros((sh, 1), jnp.float32), cp[:E - sh]], axis=0)
        sh *= 2
    excl = cp - pc
    pos0 = jnp.sum((excl + c1) * oh1t, axis=0)[None, :]
    pos1 = jnp.sum((excl + cnt1 + c2) * oh2t, axis=0)[None, :]
    pos_ref[...] = jnp.concatenate(
        [pos0, pos1], axis=1).astype(jnp.int32)

    total_padded = cp[E - 1:E, :]
    istart = lax.broadcasted_iota(
        jnp.int32, (NB, E), 0).astype(jnp.float32) * BLK
    cp_row = jnp.sum(
        (lax.broadcasted_iota(jnp.int32, (E, E), 0)
         == lax.broadcasted_iota(jnp.int32, (E, E), 1)).astype(jnp.float32)
        * cp, axis=0)[None, :]
    be_raw = jnp.sum((istart >= cp_row).astype(jnp.int32), axis=1)
    bmap_ref[...] = jnp.minimum(be_raw, E - 1)[None, :]
    bact_ref[...] = (istart[:, 0] < total_padded[0, 0]).astype(
        jnp.int32)[None, :]

    tpe = counts[:, 0] / (T * K + 1e-9)
    avg_p = jnp.mean(p, axis=0)
    lb = E * jnp.sum(tpe * avg_p)
    lse = m[:, 0] + jnp.log(denom[:, 0])
    z_loss = jnp.mean(jnp.square(lse)) * 0.001
    entropy = jnp.mean(-jnp.sum(p * jnp.log(p + 1e-9), axis=1))
    ent_loss = (jnp.log(jnp.float32(E)) - entropy) * 0.01
    usage = jnp.mean((tpe > 0.01).astype(jnp.float32))
    util_loss = (1.0 - usage) * 0.1
    aux = lb + z_loss + ent_loss + util_loss
    aux_ref[...] = jnp.broadcast_to(aux[None, None], (1, 1))


def _router(x, w_gate):
    return pl.pallas_call(
        _router_kernel,
        out_shape=(
            jax.ShapeDtypeStruct((1, P), jnp.int32),
            jax.ShapeDtypeStruct((T, 2), jnp.float32),
            jax.ShapeDtypeStruct((1, NB), jnp.int32),
            jax.ShapeDtypeStruct((1, NB), jnp.int32),
            jax.ShapeDtypeStruct((1, 1), jnp.float32),
        ),
    )(x, w_gate)


@functools.lru_cache(maxsize=None)
def _sc_dispatch_call():
    mesh = plsc.VectorSubcoreMesh(core_axis_name="c", subcore_axis_name="s")

    @functools.partial(
        pl.kernel,
        out_type=jax.ShapeDtypeStruct((P_MAX, H), jnp.float32),
        mesh=mesh,
        scratch_types=[
            pltpu.VMEM((DCH,), jnp.int32),
            pltpu.VMEM((DCH, H), jnp.float32),
            pltpu.SemaphoreType.DMA,
        ],
    )
    def _sc_dispatch(x_hbm, pos_hbm, xs_hbm, pos_v, rows_v, sem):
        wid = lax.axis_index("s") * 2 + lax.axis_index("c")
        base = wid * DCH
        tok_base = lax.rem(base, T)
        pltpu.sync_copy(pos_hbm.at[pl.ds(base, DCH)], pos_v)
        pltpu.sync_copy(x_hbm.at[pl.ds(tok_base, DCH)], rows_v)
        pltpu.async_copy(rows_v, xs_hbm.at[pos_v], sem).wait()

    return _sc_dispatch


@functools.lru_cache(maxsize=None)
def _sc_gatherback_call():
    mesh = plsc.VectorSubcoreMesh(core_axis_name="c", subcore_axis_name="s")

    @functools.partial(
        pl.kernel,
        out_type=(
            jax.ShapeDtypeStruct((T, H), jnp.float32),
            jax.ShapeDtypeStruct((T, H), jnp.float32),
        ),
        mesh=mesh,
        scratch_types=[
            pltpu.VMEM((GCH,), jnp.int32),
            pltpu.VMEM((GCH,), jnp.int32),
            pltpu.VMEM((GCH, H), jnp.float32),
            pltpu.VMEM((GCH, H), jnp.float32),
            pltpu.SemaphoreType.DMA,
        ],
    )
    def _sc_gatherback(ys_hbm, pos_hbm, y0_hbm, y1_hbm, p0_v, p1_v, r0_v,
                       r1_v, sem):
        wid = lax.axis_index("s") * 2 + lax.axis_index("c")
        base = wid * GCH
        pltpu.sync_copy(pos_hbm.at[pl.ds(base, GCH)], p0_v)
        pltpu.sync_copy(pos_hbm.at[pl.ds(T + base, GCH)], p1_v)
        cp0 = pltpu.async_copy(ys_hbm.at[p0_v], r0_v, sem)
        cp1 = pltpu.async_copy(ys_hbm.at[p1_v], r1_v, sem)
        cp0.wait()
        cp1.wait()
        pltpu.sync_copy(r0_v, y0_hbm.at[pl.ds(base, GCH)])
        pltpu.sync_copy(r1_v, y1_hbm.at[pl.ds(base, GCH)])

    return _sc_gatherback


def _xs_idx(i, bm, ba):
    return (jnp.where(ba[i] == 1, i, 0), 0)


def _w_idx(i, bm, ba):
    return (bm[i], 0, 0)


def _out_idx(i, bm, ba):
    return (jnp.where(ba[i] == 1, i, NB - 1), 0)


def _gmm_kernel(bmap_ref, bact_ref, xs_ref, wg_ref, wu_ref, wd_ref, out_ref):
    i = pl.program_id(0)

    @pl.when(bact_ref[i] == 1)
    def _():
        x = xs_ref[...]
        g = lax.dot_general(x, wg_ref[0], (((1,), (1,)), ((), ())),
                            preferred_element_type=jnp.float32)
        u = lax.dot_general(x, wu_ref[0], (((1,), (1,)), ((), ())),
                            preferred_element_type=jnp.float32)
        a = g * jax.nn.sigmoid(g) * u
        out_ref[...] = lax.dot_general(a, wd_ref[0], (((1,), (1,)), ((), ())),
                                       preferred_element_type=jnp.float32)


def _gmm(xs, Wg, Wu, Wd, bmap, bact):
    return pl.pallas_call(
        _gmm_kernel,
        grid_spec=pltpu.PrefetchScalarGridSpec(
            num_scalar_prefetch=2,
            grid=(NB,),
            in_specs=[
                pl.BlockSpec((BLK, H), _xs_idx),
                pl.BlockSpec((1, I, H), _w_idx),
                pl.BlockSpec((1, I, H), _w_idx),
                pl.BlockSpec((1, H, I), _w_idx),
            ],
            out_specs=pl.BlockSpec((BLK, H), _out_idx),
        ),
        out_shape=jax.ShapeDtypeStruct((P_MAX, H), jnp.float32),
        compiler_params=pltpu.CompilerParams(
            dimension_semantics=("arbitrary",),
        ),
    )(bmap, bact, xs, Wg, Wu, Wd)


def _shared_kernel(x_ref, wsg_ref, wsu_ref, wsd_ref, sg_ref, out_ref):
    x = x_ref[...]
    g = lax.dot_general(x, wsg_ref[...], (((1,), (1,)), ((), ())),
                        preferred_element_type=jnp.float32)
    u = lax.dot_general(x, wsu_ref[...], (((1,), (1,)), ((), ())),
                        preferred_element_type=jnp.float32)
    a = g * jax.nn.sigmoid(g) * u
    y = lax.dot_general(a, wsd_ref[...], (((1,), (1,)), ((), ())),
                        preferred_element_type=jnp.float32)
    out_ref[...] = y * jax.nn.sigmoid(sg_ref[0, 0])


def _shared(x, Wsg, Wsu, Wsd, sg):
    return pl.pallas_call(
        _shared_kernel,
        grid=(NT,),
        in_specs=[
            pl.BlockSpec((BT, H), lambda t: (t, 0)),
            pl.BlockSpec((I, H), lambda t: (0, 0)),
            pl.BlockSpec((I, H), lambda t: (0, 0)),
            pl.BlockSpec((H, I), lambda t: (0, 0)),
            pl.BlockSpec((1, 1), lambda t: (0, 0)),
        ],
        out_specs=pl.BlockSpec((BT, H), lambda t: (t, 0)),
        out_shape=jax.ShapeDtypeStruct((T, H), jnp.float32),
        compiler_params=pltpu.CompilerParams(
            dimension_semantics=("parallel",),
        ),
    )(x, Wsg, Wsu, Wsd, sg)


def _combine_kernel(y0_ref, y1_ref, sh_ref, w_ref, out_ref):
    w = w_ref[...]
    out_ref[...] = (y0_ref[...] * w[:, 0:1] + y1_ref[...] * w[:, 1:2]
                    + sh_ref[...])


def _combine(y0, y1, shr, w01):
    return pl.pallas_call(
        _combine_kernel,
        grid=(NT,),
        in_specs=[
            pl.BlockSpec((BT, H), lambda t: (t, 0)),
            pl.BlockSpec((BT, H), lambda t: (t, 0)),
            pl.BlockSpec((BT, H), lambda t: (t, 0)),
            pl.BlockSpec((BT, 2), lambda t: (t, 0)),
        ],
        out_specs=pl.BlockSpec((BT, H), lambda t: (t, 0)),
        out_shape=jax.ShapeDtypeStruct((T, H), jnp.float32),
        compiler_params=pltpu.CompilerParams(
            dimension_semantics=("parallel",),
        ),
    )(y0, y1, shr, w01)


def kernel(hidden_states, W_gate, Wg, Wu, Wd, Wsg, Wsu, Wsd, shared_gate):
    x = hidden_states.reshape(T, H)
    pos, w01, bmap, bact, aux = _router(x, W_gate)
    pos1d = pos.reshape(P)
    xs = _sc_dispatch_call()(x, pos1d)
    ys = _gmm(xs, Wg, Wu, Wd, bmap.reshape(NB), bact.reshape(NB))
    shr = _shared(x, Wsg, Wsu, Wsd, shared_gate.reshape(1, 1))
    y0, y1 = _sc_gatherback_call()(ys, pos1d)
    out = _combine(y0, y1, shr, w01)
    return out.reshape(B, S, H), aux[0, 0]


# speedup vs baseline: 1.3142x; 1.0169x over previous
import functools

import jax
import jax.numpy as jnp
from jax import lax
from jax.experimental import pallas as pl
from jax.experimental.pallas import tpu as pltpu
from jax.experimental.pallas import tpu_sc as plsc

B, S, H = 1, 2048, 768
I = 1536
E = 8
K = 2
T = B * S
P = K * T

BLK = 256
NB = P // BLK + E
P_MAX = NB * BLK

NW = 32
GCH = T // NW

BT = 1024
NT = T // BT


def _router_kernel(x_ref, wg_ref, pos_ref, w01_ref, bmap_ref, bact_ref,
                   aux_ref):
    x = x_ref[...]
    wg = wg_ref[...]
    logits = lax.dot_general(x, wg, (((1,), (1,)), ((), ())),
                             preferred_element_type=jnp.float32)
    m = jnp.max(logits, axis=1, keepdims=True)
    ex = jnp.exp(logits - m)
    denom = jnp.sum(ex, axis=1, keepdims=True)
    p = ex / denom

    i1 = jnp.argmax(p, axis=1)
    cols = lax.broadcasted_iota(jnp.int32, (T, E), 1)
    oh1 = (cols == i1[:, None]).astype(jnp.float32)
    v1 = jnp.sum(p * oh1, axis=1, keepdims=True)
    p2 = jnp.where(cols == i1[:, None], -jnp.inf, p)
    i2 = jnp.argmax(p2, axis=1)
    oh2 = (cols == i2[:, None]).astype(jnp.float32)
    v2 = jnp.sum(p * oh2, axis=1, keepdims=True)

    s = v1 + v2 + 1e-9
    w01_ref[...] = jnp.concatenate([v1 / s, v2 / s], axis=1)

    rows = lax.broadcasted_iota(jnp.int32, (E, T), 0)
    oh1t = (rows == i1[None, :]).astype(jnp.float32)
    oh2t = (rows == i2[None, :]).astype(jnp.float32)

    def _lane_cumsum_excl(a):
        c = a
        sh = 1
        while sh < T:
            c = c + jnp.concatenate(
                [jnp.zeros((E, sh), jnp.float32), c[:, :T - sh]], axis=1)
            sh *= 2
        return c - a

    c1 = _lane_cumsum_excl(oh1t)
    c2 = _lane_cumsum_excl(oh2t)
    cnt1 = jnp.sum(oh1t, axis=1, keepdims=True)
    counts = cnt1 + jnp.sum(oh2t, axis=1, keepdims=True)
    pc = jnp.ceil(counts / BLK) * BLK
    cp = pc
    sh = 1
    while sh < E:
        cp = cp + jnp.concatenate(
            [jnp.zeros((sh, 1), jnp.float32), cp[:E - sh]], axis=0)
        sh *= 2
    excl = cp - pc
    pos0 = jnp.sum((excl + c1) * oh1t, axis=0)[None, :]
    pos1 = jnp.sum((excl + cnt1 + c2) * oh2t, axis=0)[None, :]
    pos_ref[...] = jnp.concatenate(
        [pos0, pos1], axis=1).astype(jnp.int32)

    total_padded = cp[E - 1:E, :]
    istart = lax.broadcasted_iota(
        jnp.int32, (NB, E), 0).astype(jnp.float32) * BLK
    cp_row = jnp.sum(
        (lax.broadcasted_iota(jnp.int32, (E, E), 0)
         == lax.broadcasted_iota(jnp.int32, (E, E), 1)).astype(jnp.float32)
        * cp, axis=0)[None, :]
    be_raw = jnp.sum((istart >= cp_row).astype(jnp.int32), axis=1)
    bmap_ref[...] = jnp.minimum(be_raw, E - 1)[None, :]
    bact_ref[...] = (istart[:, 0] < total_padded[0, 0]).astype(
        jnp.int32)[None, :]

    tpe = counts[:, 0] / (T * K + 1e-9)
    avg_p = jnp.mean(p, axis=0)
    lb = E * jnp.sum(tpe * avg_p)
    lse = m[:, 0] + jnp.log(denom[:, 0])
    z_loss = jnp.mean(jnp.square(lse)) * 0.001
    entropy = jnp.mean(-jnp.sum(p * jnp.log(p + 1e-9), axis=1))
    ent_loss = (jnp.log(jnp.float32(E)) - entropy) * 0.01
    usage = jnp.mean((tpe > 0.01).astype(jnp.float32))
    util_loss = (1.0 - usage) * 0.1
    aux = lb + z_loss + ent_loss + util_loss
    aux_ref[...] = jnp.broadcast_to(aux[None, None], (1, 1))


def _router(x, w_gate):
    return pl.pallas_call(
        _router_kernel,
        out_shape=(
            jax.ShapeDtypeStruct((1, P), jnp.int32),
            jax.ShapeDtypeStruct((T, 2), jnp.float32),
            jax.ShapeDtypeStruct((1, NB), jnp.int32),
            jax.ShapeDtypeStruct((1, NB), jnp.int32),
            jax.ShapeDtypeStruct((1, 1), jnp.float32),
        ),
    )(x, w_gate)


@functools.lru_cache(maxsize=None)
def _sc_dispatch_call():
    mesh = plsc.VectorSubcoreMesh(core_axis_name="c", subcore_axis_name="s")

    @functools.partial(
        pl.kernel,
        out_type=jax.ShapeDtypeStruct((P_MAX, H), jnp.float32),
        mesh=mesh,
        scratch_types=[
            pltpu.VMEM((GCH,), jnp.int32),
            pltpu.VMEM((GCH,), jnp.int32),
            pltpu.VMEM((GCH, H), jnp.float32),
            pltpu.SemaphoreType.DMA,
        ],
    )
    def _sc_dispatch(x_hbm, pos_hbm, xs_hbm, p0_v, p1_v, rows_v, sem):
        wid = lax.axis_index("s") * 2 + lax.axis_index("c")
        base = wid * GCH
        pltpu.sync_copy(pos_hbm.at[pl.ds(base, GCH)], p0_v)
        pltpu.sync_copy(pos_hbm.at[pl.ds(T + base, GCH)], p1_v)
        pltpu.sync_copy(x_hbm.at[pl.ds(base, GCH)], rows_v)
        cp0 = pltpu.async_copy(rows_v, xs_hbm.at[p0_v], sem)
        cp1 = pltpu.async_copy(rows_v, xs_hbm.at[p1_v], sem)
        cp0.wait()
        cp1.wait()

    return _sc_dispatch


@functools.lru_cache(maxsize=None)
def _sc_gatherback_call():
    mesh = plsc.VectorSubcoreMesh(core_axis_name="c", subcore_axis_name="s")

    @functools.partial(
        pl.kernel,
        out_type=(
            jax.ShapeDtypeStruct((T, H), jnp.float32),
            jax.ShapeDtypeStruct((T, H), jnp.float32),
        ),
        mesh=mesh,
        scratch_types=[
            pltpu.VMEM((GCH,), jnp.int32),
            pltpu.VMEM((GCH,), jnp.int32),
            pltpu.VMEM((GCH, H), jnp.float32),
            pltpu.VMEM((GCH, H), jnp.float32),
            pltpu.SemaphoreType.DMA,
        ],
    )
    def _sc_gatherback(ys_hbm, pos_hbm, y0_hbm, y1_hbm, p0_v, p1_v, r0_v,
                       r1_v, sem):
        wid = lax.axis_index("s") * 2 + lax.axis_index("c")
        base = wid * GCH
        pltpu.sync_copy(pos_hbm.at[pl.ds(base, GCH)], p0_v)
        pltpu.sync_copy(pos_hbm.at[pl.ds(T + base, GCH)], p1_v)
        cp0 = pltpu.async_copy(ys_hbm.at[p0_v], r0_v, sem)
        cp1 = pltpu.async_copy(ys_hbm.at[p1_v], r1_v, sem)
        cp0.wait()
        cp1.wait()
        pltpu.sync_copy(r0_v, y0_hbm.at[pl.ds(base, GCH)])
        pltpu.sync_copy(r1_v, y1_hbm.at[pl.ds(base, GCH)])

    return _sc_gatherback


def _xs_idx(i, bm, ba):
    return (jnp.where(ba[i] == 1, i, 0), 0)


def _w_idx(i, bm, ba):
    return (bm[i], 0, 0)


def _out_idx(i, bm, ba):
    return (jnp.where(ba[i] == 1, i, NB - 1), 0)


def _gmm_kernel(bmap_ref, bact_ref, xs_ref, wg_ref, wu_ref, wd_ref, out_ref):
    i = pl.program_id(0)

    @pl.when(bact_ref[i] == 1)
    def _():
        x = xs_ref[...]
        g = lax.dot_general(x, wg_ref[0], (((1,), (1,)), ((), ())),
                            preferred_element_type=jnp.float32)
        u = lax.dot_general(x, wu_ref[0], (((1,), (1,)), ((), ())),
                            preferred_element_type=jnp.float32)
        a = g * jax.nn.sigmoid(g) * u
        out_ref[...] = lax.dot_general(a, wd_ref[0], (((1,), (1,)), ((), ())),
                                       preferred_element_type=jnp.float32)


def _gmm(xs, Wg, Wu, Wd, bmap, bact):
    return pl.pallas_call(
        _gmm_kernel,
        grid_spec=pltpu.PrefetchScalarGridSpec(
            num_scalar_prefetch=2,
            grid=(NB,),
            in_specs=[
                pl.BlockSpec((BLK, H), _xs_idx),
                pl.BlockSpec((1, I, H), _w_idx),
                pl.BlockSpec((1, I, H), _w_idx),
                pl.BlockSpec((1, H, I), _w_idx),
            ],
            out_specs=pl.BlockSpec((BLK, H), _out_idx),
        ),
        out_shape=jax.ShapeDtypeStruct((P_MAX, H), jnp.float32),
        compiler_params=pltpu.CompilerParams(
            dimension_semantics=("arbitrary",),
        ),
    )(bmap, bact, xs, Wg, Wu, Wd)


def _shared_kernel(x_ref, wsg_ref, wsu_ref, wsd_ref, sg_ref, out_ref):
    x = x_ref[...]
    g = lax.dot_general(x, wsg_ref[...], (((1,), (1,)), ((), ())),
                        preferred_element_type=jnp.float32)
    u = lax.dot_general(x, wsu_ref[...], (((1,), (1,)), ((), ())),
                        preferred_element_type=jnp.float32)
    a = g * jax.nn.sigmoid(g) * u
    y = lax.dot_general(a, wsd_ref[...], (((1,), (1,)), ((), ())),
                        preferred_element_type=jnp.float32)
    out_ref[...] = y * jax.nn.sigmoid(sg_ref[0, 0])


def _shared(x, Wsg, Wsu, Wsd, sg):
    return pl.pallas_call(
        _shared_kernel,
        grid=(NT,),
        in_specs=[
            pl.BlockSpec((BT, H), lambda t: (t, 0)),
            pl.BlockSpec((I, H), lambda t: (0, 0)),
            pl.BlockSpec((I, H), lambda t: (0, 0)),
            pl.BlockSpec((H, I), lambda t: (0, 0)),
            pl.BlockSpec((1, 1), lambda t: (0, 0)),
        ],
        out_specs=pl.BlockSpec((BT, H), lambda t: (t, 0)),
        out_shape=jax.ShapeDtypeStruct((T, H), jnp.float32),
        compiler_params=pltpu.CompilerParams(
            dimension_semantics=("parallel",),
        ),
    )(x, Wsg, Wsu, Wsd, sg)


def _combine_kernel(y0_ref, y1_ref, sh_ref, w_ref, out_ref):
    w = w_ref[...]
    out_ref[...] = (y0_ref[...] * w[:, 0:1] + y1_ref[...] * w[:, 1:2]
                    + sh_ref[...])


def _combine(y0, y1, shr, w01):
    return pl.pallas_call(
        _combine_kernel,
        grid=(NT,),
        in_specs=[
            pl.BlockSpec((BT, H), lambda t: (t, 0)),
            pl.BlockSpec((BT, H), lambda t: (t, 0)),
            pl.BlockSpec((BT, H), lambda t: (t, 0)),
            pl.BlockSpec((BT, 2), lambda t: (t, 0)),
        ],
        out_specs=pl.BlockSpec((BT, H), lambda t: (t, 0)),
        out_shape=jax.ShapeDtypeStruct((T, H), jnp.float32),
        compiler_params=pltpu.CompilerParams(
            dimension_semantics=("parallel",),
        ),
    )(y0, y1, shr, w01)


def kernel(hidden_states, W_gate, Wg, Wu, Wd, Wsg, Wsu, Wsd, shared_gate):
    x = hidden_states.reshape(T, H)
    pos, w01, bmap, bact, aux = _router(x, W_gate)
    pos1d = pos.reshape(P)
    xs = _sc_dispatch_call()(x, pos1d)
    ys = _gmm(xs, Wg, Wu, Wd, bmap.reshape(NB), bact.reshape(NB))
    shr = _shared(x, Wsg, Wsu, Wsd, shared_gate.reshape(1, 1))
    y0, y1 = _sc_gatherback_call()(ys, pos1d)
    out = _combine(y0, y1, shr, w01)
    return out.reshape(B, S, H), aux[0, 0]
